# baseline (speedup 1.0000x reference)
import math

import jax
import jax.numpy as jnp
from jax import lax
from jax.experimental import pallas as pl
from jax.experimental.pallas import tpu as pltpu
from jax.experimental.pallas import tpu_sc as plsc

_N_LEVELS = 16
_F = 2
_T = 2 ** 19
_BASE_RES = 16
_MAX_RES = 512
_B = 524288
_SCALE = math.exp(math.log(_MAX_RES / _BASE_RES) / (_N_LEVELS - 1))
_RES = []
_OFF = []
_tot = 0
for _i in range(_N_LEVELS):
    _OFF.append(_tot)
    _r = math.floor(_BASE_RES * _SCALE ** _i)
    _RES.append(_r)
    _tot += min(_T, (_r + 1) ** 3)
_OFF.append(_tot)
_N_TOTAL = _tot
_SIZES = [_OFF[i + 1] - _OFF[i] for i in range(_N_LEVELS)]
_P1 = 2654435761
_P2 = 805459861
_OUT_D = 3 + 2 * _N_LEVELS

_NW = 32
_PW = _B // _NW
_C = 1024
_NCH = _PW // _C
_G = _C // 16

_N_RES_LEVELS = 1
_TAB_WORDS = _OFF[_N_RES_LEVELS] * _F


def _hash_corners(gx, gy, gz):
    hx = (gx, gx + jnp.uint32(1))
    hy0 = gy * jnp.uint32(_P1)
    hy = (hy0, hy0 + jnp.uint32(_P1))
    hz0 = gz * jnp.uint32(_P2)
    hz = (hz0, hz0 + jnp.uint32(_P2))
    hyz = (hy[0] ^ hz[0], hy[0] ^ hz[1], hy[1] ^ hz[0], hy[1] ^ hz[1])
    return tuple(hx[(k >> 2) & 1] ^ hyz[k & 3] for k in range(8))


def _mod_level(h, size, off):
    if size == _T:
        r = h & jnp.uint32(_T - 1)
    else:
        hf = h.astype(jnp.float32)
        q = (hf * jnp.float32(1.0 / size)).astype(jnp.int32)
        qu = lax.bitcast_convert_type(q, jnp.uint32)
        r = h - qu * jnp.uint32(size)
        ri = lax.bitcast_convert_type(r, jnp.int32)
        r = jnp.where(ri < 0, r + jnp.uint32(size), r)
        r = jnp.where(r >= jnp.uint32(size), r - jnp.uint32(size), r)
    hidx = lax.bitcast_convert_type(r, jnp.int32) + off
    return hidx + hidx


def _prep(v, res):
    vn = jnp.minimum(jnp.maximum(v, jnp.float32(0.0)), jnp.float32(1.0))
    pos = vn * jnp.float32(res)
    gi = pos.astype(jnp.int32)
    gi = jnp.minimum(gi, jnp.int32(res - 1))
    fr = pos - gi.astype(jnp.float32)
    return gi.astype(jnp.uint32), fr


def _corner_w(fx, fy, fz):
    one = jnp.float32(1.0)
    wx = (one - fx, fx)
    wy = (one - fy, fy)
    wz = (one - fz, fz)
    return tuple((wx[(k >> 2) & 1] * wy[(k >> 1) & 1]) * wz[k & 1]
                 for k in range(8))


def _body(xyzf, emb, out, xyzb, f0x, f0y, f0z, f1x, f1y, f1z,
          idx0, idx1, vals0, vals1, tab, ob, sem0, sem1):
    wid = lax.axis_index("s") * 2 + lax.axis_index("c")
    iota = lax.iota(jnp.int32, 16)
    iota3 = iota * 3
    one_i = jnp.full((16,), 1, jnp.int32)
    fbufs = ((f0x, f0y, f0z), (f1x, f1y, f1z))
    ibufs = (idx0, idx1)
    vbufs = (vals0, vals1)
    sems = (sem0, sem1)

    pltpu.sync_copy(emb.at[pl.ds(0, _TAB_WORDS)], tab)

    def chunk_body(ch, carry):
        base = wid * _PW + ch * _C
        pltpu.sync_copy(xyzf.at[pl.ds(base * 3, 3 * _C)], xyzb)

        def fused(g, c2):
            g48 = iota3 + g * 48
            x = plsc.load_gather(xyzb, [g48])
            y = plsc.load_gather(xyzb, [g48 + one_i])
            z = plsc.load_gather(xyzb, [g48 + one_i + one_i])
            s16o = g * 16
            ob[0, pl.ds(s16o, 16)] = x
            ob[1, pl.ds(s16o, 16)] = y
            ob[2, pl.ds(s16o, 16)] = z
            for l in range(_N_RES_LEVELS):
                gx, fx = _prep(x, _RES[l])
                gy, fy = _prep(y, _RES[l])
                gz, fz = _prep(z, _RES[l])
                hs = _hash_corners(gx, gy, gz)
                ws = _corner_w(fx, fy, fz)
                acc0 = jnp.zeros((16,), jnp.float32)
                acc1 = jnp.zeros((16,), jnp.float32)
                for k in range(8):
                    h2 = _mod_level(hs[k], _SIZES[l], _OFF[l])
                    v0 = plsc.load_gather(tab, [h2])
                    v1 = plsc.load_gather(tab, [h2 + one_i])
                    acc0 = acc0 + ws[k] * v0
                    acc1 = acc1 + ws[k] * v1
                ob[3 + 2 * l, pl.ds(s16o, 16)] = acc0
                ob[4 + 2 * l, pl.ds(s16o, 16)] = acc1
            return c2

        lax.fori_loop(0, _G, fused, 0)

        def make_p1(l):
            res = _RES[l]
            off = _OFF[l]
            size = _SIZES[l]
            fxb, fyb, fzb = fbufs[l % 2]
            idxr = ibufs[l % 2]

            def p1(g, c2):
                s16o = g * 16
                g48 = iota3 + g * 48
                x = plsc.load_gather(xyzb, [g48])
                y = plsc.load_gather(xyzb, [g48 + one_i])
                z = plsc.load_gather(xyzb, [g48 + one_i + one_i])
                gx, fx = _prep(x, res)
                gy, fy = _prep(y, res)
                gz, fz = _prep(z, res)
                fxb[pl.ds(s16o, 16)] = fx
                fyb[pl.ds(s16o, 16)] = fy
                fzb[pl.ds(s16o, 16)] = fz
                hs = _hash_corners(gx, gy, gz)
                for k in range(8):
                    h2 = _mod_level(hs[k], size, off)
                    idxr[pl.ds(2 * k * _C + s16o, 16)] = h2
                    idxr[pl.ds((2 * k + 1) * _C + s16o, 16)] = h2 + one_i
                return c2

            return p1

        def make_p2(l):
            fxb, fyb, fzb = fbufs[l % 2]
            vals = vbufs[l % 2]

            def p2(g, c2):
                s16 = pl.ds(g * 16, 16)
                ws = _corner_w(fxb[s16], fyb[s16], fzb[s16])
                acc0 = jnp.zeros((16,), jnp.float32)
                acc1 = jnp.zeros((16,), jnp.float32)
                r0 = g * 16 + iota
                for k in range(8):
                    v0 = plsc.load_gather(vals, [r0 + 2 * k * _C])
                    v1 = plsc.load_gather(vals, [r0 + (2 * k + 1) * _C])
                    acc0 = acc0 + ws[k] * v0
                    acc1 = acc1 + ws[k] * v1
                s16o = g * 16
                ob[3 + 2 * l, pl.ds(s16o, 16)] = acc0
                ob[4 + 2 * l, pl.ds(s16o, 16)] = acc1
                return c2

            return p2

        def start_gather(l):
            return pltpu.async_copy(emb.at[ibufs[l % 2]], vbufs[l % 2],
                                    sems[l % 2])

        l0 = _N_RES_LEVELS
        lax.fori_loop(0, _G, make_p1(l0), 0)
        handle = start_gather(l0)
        for l in range(l0, _N_LEVELS):
            nxt = None
            if l + 1 < _N_LEVELS:
                lax.fori_loop(0, _G, make_p1(l + 1), 0)
                nxt = start_gather(l + 1)
            handle.wait()
            lax.fori_loop(0, _G, make_p2(l), 0)
            handle = nxt

        pltpu.sync_copy(ob, out.at[:, pl.ds(base, _C)])
        return carry

    lax.fori_loop(0, _NCH, chunk_body, 0)


_sc_call = pl.kernel(
    _body,
    out_type=jax.ShapeDtypeStruct((_OUT_D, _B), jnp.float32),
    mesh=plsc.VectorSubcoreMesh(core_axis_name="c", subcore_axis_name="s"),
    compiler_params=pltpu.CompilerParams(
        needs_layout_passes=False, use_tc_tiling_on_sc=False),
    scratch_types=[
        pltpu.VMEM((3 * _C,), jnp.float32),
        pltpu.VMEM((_C,), jnp.float32),
        pltpu.VMEM((_C,), jnp.float32),
        pltpu.VMEM((_C,), jnp.float32),
        pltpu.VMEM((_C,), jnp.float32),
        pltpu.VMEM((_C,), jnp.float32),
        pltpu.VMEM((_C,), jnp.float32),
        pltpu.VMEM((2 * 8 * _C,), jnp.int32),
        pltpu.VMEM((2 * 8 * _C,), jnp.int32),
        pltpu.VMEM((2 * 8 * _C,), jnp.float32),
        pltpu.VMEM((2 * 8 * _C,), jnp.float32),
        pltpu.VMEM((_TAB_WORDS,), jnp.float32),
        pltpu.VMEM((_OUT_D, _C), jnp.float32),
        pltpu.SemaphoreType.DMA,
        pltpu.SemaphoreType.DMA,
    ],
)


@jax.jit
def kernel(xyz, embeddings):
    xyz_flat = xyz.reshape(-1)
    emb_flat = embeddings.reshape(-1)
    out_t = _sc_call(xyz_flat, emb_flat)
    return out_t.T + jnp.float32(0.0)


# speedup vs baseline: 1.0190x; 1.0190x over previous
import math

import jax
import jax.numpy as jnp
from jax import lax
from jax.experimental import pallas as pl
from jax.experimental.pallas import tpu as pltpu
from jax.experimental.pallas import tpu_sc as plsc

_N_LEVELS = 16
_F = 2
_T = 2 ** 19
_BASE_RES = 16
_MAX_RES = 512
_B = 524288
_SCALE = math.exp(math.log(_MAX_RES / _BASE_RES) / (_N_LEVELS - 1))
_RES = []
_OFF = []
_tot = 0
for _i in range(_N_LEVELS):
    _OFF.append(_tot)
    _r = math.floor(_BASE_RES * _SCALE ** _i)
    _RES.append(_r)
    _tot += min(_T, (_r + 1) ** 3)
_OFF.append(_tot)
_N_TOTAL = _tot
_SIZES = [_OFF[i + 1] - _OFF[i] for i in range(_N_LEVELS)]
_P1 = 2654435761
_P2 = 805459861
_OUT_D = 3 + 2 * _N_LEVELS
_N_PLANES = 2 * _N_LEVELS

_NW = 32
_PW = _B // _NW
_C = 1024
_NCH = _PW // _C
_G = _C // 16

_N_RES_LEVELS = 1
_TAB_WORDS = _OFF[_N_RES_LEVELS] * _F

_TR = 4096


def _hash_corners(gx, gy, gz):
    hx = (gx, gx + jnp.uint32(1))
    hy0 = gy * jnp.uint32(_P1)
    hy = (hy0, hy0 + jnp.uint32(_P1))
    hz0 = gz * jnp.uint32(_P2)
    hz = (hz0, hz0 + jnp.uint32(_P2))
    hyz = (hy[0] ^ hz[0], hy[0] ^ hz[1], hy[1] ^ hz[0], hy[1] ^ hz[1])
    return tuple(hx[(k >> 2) & 1] ^ hyz[k & 3] for k in range(8))


def _mod_level(h, size, off):
    if size == _T:
        r = h & jnp.uint32(_T - 1)
    else:
        hf = h.astype(jnp.float32)
        q = (hf * jnp.float32(1.0 / size)).astype(jnp.int32)
        qu = lax.bitcast_convert_type(q, jnp.uint32)
        r = h - qu * jnp.uint32(size)
        ri = lax.bitcast_convert_type(r, jnp.int32)
        r = jnp.where(ri < 0, r + jnp.uint32(size), r)
        r = jnp.where(r >= jnp.uint32(size), r - jnp.uint32(size), r)
    hidx = lax.bitcast_convert_type(r, jnp.int32) + off
    return hidx + hidx


def _prep(v, res):
    vn = jnp.minimum(jnp.maximum(v, jnp.float32(0.0)), jnp.float32(1.0))
    pos = vn * jnp.float32(res)
    gi = pos.astype(jnp.int32)
    gi = jnp.minimum(gi, jnp.int32(res - 1))
    fr = pos - gi.astype(jnp.float32)
    return gi.astype(jnp.uint32), fr


def _corner_w(fx, fy, fz):
    one = jnp.float32(1.0)
    wx = (one - fx, fx)
    wy = (one - fy, fy)
    wz = (one - fz, fz)
    return tuple((wx[(k >> 2) & 1] * wy[(k >> 1) & 1]) * wz[k & 1]
                 for k in range(8))


def _body(*refs):
    xh, yh, zh, emb = refs[0:4]
    outs = refs[4:4 + _N_PLANES]
    (xb, yb, zb, f0x, f0y, f0z, f1x, f1y, f1z,
     idx0, idx1, vals0, vals1, tab, ob, sem0, sem1) = refs[4 + _N_PLANES:]
    wid = lax.axis_index("s") * 2 + lax.axis_index("c")
    iota = lax.iota(jnp.int32, 16)
    one_i = jnp.full((16,), 1, jnp.int32)
    fbufs = ((f0x, f0y, f0z), (f1x, f1y, f1z))
    ibufs = (idx0, idx1)
    vbufs = (vals0, vals1)
    sems = (sem0, sem1)

    pltpu.sync_copy(emb.at[pl.ds(0, _TAB_WORDS)], tab)

    def chunk_body(ch, carry):
        base = wid * _PW + ch * _C
        pltpu.sync_copy(xh.at[pl.ds(base, _C)], xb)
        pltpu.sync_copy(yh.at[pl.ds(base, _C)], yb)
        pltpu.sync_copy(zh.at[pl.ds(base, _C)], zb)

        def fused(g, c2):
            s16 = pl.ds(g * 16, 16)
            x = xb[s16]
            y = yb[s16]
            z = zb[s16]
            for l in range(_N_RES_LEVELS):
                gx, fx = _prep(x, _RES[l])
                gy, fy = _prep(y, _RES[l])
                gz, fz = _prep(z, _RES[l])
                hs = _hash_corners(gx, gy, gz)
                ws = _corner_w(fx, fy, fz)
                acc0 = jnp.zeros((16,), jnp.float32)
                acc1 = jnp.zeros((16,), jnp.float32)
                for k in range(8):
                    h2 = _mod_level(hs[k], _SIZES[l], _OFF[l])
                    v0 = plsc.load_gather(tab, [h2])
                    v1 = plsc.load_gather(tab, [h2 + one_i])
                    acc0 = acc0 + ws[k] * v0
                    acc1 = acc1 + ws[k] * v1
                ob[2 * l, s16] = acc0
                ob[2 * l + 1, s16] = acc1
            return c2

        lax.fori_loop(0, _G, fused, 0)

        def make_p1(l):
            res = _RES[l]
            off = _OFF[l]
            size = _SIZES[l]
            fxb, fyb, fzb = fbufs[l % 2]
            idxr = ibufs[l % 2]

            def p1(g, c2):
                s16 = pl.ds(g * 16, 16)
                s16o = g * 16
                gx, fx = _prep(xb[s16], res)
                gy, fy = _prep(yb[s16], res)
                gz, fz = _prep(zb[s16], res)
                fxb[s16] = fx
                fyb[s16] = fy
                fzb[s16] = fz
                hs = _hash_corners(gx, gy, gz)
                for k in range(8):
                    h2 = _mod_level(hs[k], size, off)
                    idxr[pl.ds(2 * k * _C + s16o, 16)] = h2
                    idxr[pl.ds((2 * k + 1) * _C + s16o, 16)] = h2 + one_i
                return c2

            return p1

        def make_p2(l):
            fxb, fyb, fzb = fbufs[l % 2]
            vals = vbufs[l % 2]

            def p2(g, c2):
                s16 = pl.ds(g * 16, 16)
                ws = _corner_w(fxb[s16], fyb[s16], fzb[s16])
                acc0 = jnp.zeros((16,), jnp.float32)
                acc1 = jnp.zeros((16,), jnp.float32)
                r0 = g * 16 + iota
                for k in range(8):
                    v0 = plsc.load_gather(vals, [r0 + 2 * k * _C])
                    v1 = plsc.load_gather(vals, [r0 + (2 * k + 1) * _C])
                    acc0 = acc0 + ws[k] * v0
                    acc1 = acc1 + ws[k] * v1
                ob[2 * l, s16] = acc0
                ob[2 * l + 1, s16] = acc1
                return c2

            return p2

        def start_gather(l):
            return pltpu.async_copy(emb.at[ibufs[l % 2]], vbufs[l % 2],
                                    sems[l % 2])

        l0 = _N_RES_LEVELS
        lax.fori_loop(0, _G, make_p1(l0), 0)
        handle = start_gather(l0)
        for l in range(l0, _N_LEVELS):
            nxt = None
            if l + 1 < _N_LEVELS:
                lax.fori_loop(0, _G, make_p1(l + 1), 0)
                nxt = start_gather(l + 1)
            handle.wait()
            lax.fori_loop(0, _G, make_p2(l), 0)
            handle = nxt

        for c in range(_N_PLANES):
            pltpu.sync_copy(ob.at[c], outs[c].at[pl.ds(base, _C)])
        return carry

    lax.fori_loop(0, _NCH, chunk_body, 0)


_sc_call = pl.kernel(
    _body,
    out_type=[jax.ShapeDtypeStruct((_B,), jnp.float32)] * _N_PLANES,
    mesh=plsc.VectorSubcoreMesh(core_axis_name="c", subcore_axis_name="s"),
    compiler_params=pltpu.CompilerParams(
        needs_layout_passes=False, use_tc_tiling_on_sc=False),
    scratch_types=[
        pltpu.VMEM((_C,), jnp.float32),
        pltpu.VMEM((_C,), jnp.float32),
        pltpu.VMEM((_C,), jnp.float32),
        pltpu.VMEM((_C,), jnp.float32),
        pltpu.VMEM((_C,), jnp.float32),
        pltpu.VMEM((_C,), jnp.float32),
        pltpu.VMEM((_C,), jnp.float32),
        pltpu.VMEM((_C,), jnp.float32),
        pltpu.VMEM((_C,), jnp.float32),
        pltpu.VMEM((2 * 8 * _C,), jnp.int32),
        pltpu.VMEM((2 * 8 * _C,), jnp.int32),
        pltpu.VMEM((2 * 8 * _C,), jnp.float32),
        pltpu.VMEM((2 * 8 * _C,), jnp.float32),
        pltpu.VMEM((_TAB_WORDS,), jnp.float32),
        pltpu.VMEM((_N_PLANES, _C), jnp.float32),
        pltpu.SemaphoreType.DMA,
        pltpu.SemaphoreType.DMA,
    ],
)


def _assemble_body(*refs):
    in_refs = refs[:_OUT_D]
    o_ref = refs[_OUT_D]
    stacked = jnp.stack([r[...] for r in in_refs])
    o_ref[...] = stacked.T


_assemble = pl.pallas_call(
    _assemble_body,
    grid=(_B // _TR,),
    in_specs=[pl.BlockSpec((_TR,), lambda i: (i,))] * _OUT_D,
    out_specs=pl.BlockSpec((_TR, _OUT_D), lambda i: (i, 0)),
    out_shape=jax.ShapeDtypeStruct((_B, _OUT_D), jnp.float32),
)


@jax.jit
def kernel(xyz, embeddings):
    x = xyz[:, 0]
    y = xyz[:, 1]
    z = xyz[:, 2]
    emb_flat = embeddings.reshape(-1)
    planes = _sc_call(x, y, z, emb_flat)
    return _assemble(x, y, z, *planes)


# speedup vs baseline: 3.4347x; 3.3705x over previous
import math

import jax
import jax.numpy as jnp
from jax import lax
from jax.experimental import pallas as pl
from jax.experimental.pallas import tpu as pltpu
from jax.experimental.pallas import tpu_sc as plsc

_N_LEVELS = 16
_F = 2
_T = 2 ** 19
_BASE_RES = 16
_MAX_RES = 512
_B = 524288
_SCALE = math.exp(math.log(_MAX_RES / _BASE_RES) / (_N_LEVELS - 1))
_RES = []
_OFF = []
_tot = 0
for _i in range(_N_LEVELS):
    _OFF.append(_tot)
    _r = math.floor(_BASE_RES * _SCALE ** _i)
    _RES.append(_r)
    _tot += min(_T, (_r + 1) ** 3)
_OFF.append(_tot)
_N_TOTAL = _tot
_SIZES = [_OFF[i + 1] - _OFF[i] for i in range(_N_LEVELS)]
_P1 = 2654435761
_P2 = 805459861
_OUT_D = 3 + 2 * _N_LEVELS
_N_PLANES = 2 * _N_LEVELS

_NW = 32
_PW = _B // _NW
_C = 512
_NCH = _PW // _C
_G = _C // 16

_NROW_PAD = 5262480
_CP = 2048
_C2 = 8 * 512
_N_RES_LEVELS = 1
_TAB_ROWS = _OFF[_N_RES_LEVELS]
_TAB_P1 = 4924
_TAB_WORDS = _TAB_P1 + _TAB_ROWS + 11

_TR = 4096


def _hash_corners(gx, gy, gz):
    hx = (gx, gx + jnp.uint32(1))
    hy0 = gy * jnp.uint32(_P1)
    hy = (hy0, hy0 + jnp.uint32(_P1))
    hz0 = gz * jnp.uint32(_P2)
    hz = (hz0, hz0 + jnp.uint32(_P2))
    hyz = (hy[0] ^ hz[0], hy[0] ^ hz[1], hy[1] ^ hz[0], hy[1] ^ hz[1])
    return tuple(hx[(k >> 2) & 1] ^ hyz[k & 3] for k in range(8))


def _mod_level(h, size, off):
    if size == _T:
        r = h & jnp.uint32(_T - 1)
    else:
        hf = h.astype(jnp.float32)
        q = (hf * jnp.float32(1.0 / size)).astype(jnp.int32)
        qu = lax.bitcast_convert_type(q, jnp.uint32)
        r = h - qu * jnp.uint32(size)
        ri = lax.bitcast_convert_type(r, jnp.int32)
        r = jnp.where(ri < 0, r + jnp.uint32(size), r)
        r = jnp.where(r >= jnp.uint32(size), r - jnp.uint32(size), r)
    return lax.bitcast_convert_type(r, jnp.int32) + off


def _prep(v, res):
    vn = jnp.minimum(jnp.maximum(v, jnp.float32(0.0)), jnp.float32(1.0))
    pos = vn * jnp.float32(res)
    gi = pos.astype(jnp.int32)
    gi = jnp.minimum(gi, jnp.int32(res - 1))
    fr = pos - gi.astype(jnp.float32)
    return gi.astype(jnp.uint32), fr


def _corner_w(fx, fy, fz):
    one = jnp.float32(1.0)
    wx = (one - fx, fx)
    wy = (one - fy, fy)
    wz = (one - fz, fz)
    return tuple((wx[(k >> 2) & 1] * wy[(k >> 1) & 1]) * wz[k & 1]
                 for k in range(8))


def _body(*refs):
    xcat, emb = refs[0:2]
    outs = refs[2:2 + _N_PLANES]
    out_tab = refs[2 + _N_PLANES]
    (xb, yb, zb, f0x, f0y, f0z, f1x, f1y, f1z,
     idx0, idx1, vals0, vals1, tab, itlb, ob, sem0, sem1) = refs[3 + _N_PLANES:]
    cid = lax.axis_index("c")
    sid = lax.axis_index("s")
    wid = sid * 2 + cid
    iota = lax.iota(jnp.int32, 16)
    one_i = jnp.full((16,), 1, jnp.int32)
    fbufs = ((f0x, f0y, f0z), (f1x, f1y, f1z))
    ibufs = (idx0, idx1)
    vbufs = (vals0, vals1)
    sems = (sem0, sem1)

    pltpu.sync_copy(emb.at[pl.ds(0, _TAB_ROWS)], tab.at[pl.ds(0, _TAB_ROWS)])
    pltpu.sync_copy(emb.at[pl.ds(_N_TOTAL - 4, _TAB_ROWS + 8)],
                    tab.at[pl.ds(_TAB_P1 - 4, _TAB_ROWS + 8)])

    region = cid * _NROW_PAD
    n_steps = (_N_TOTAL + 16 * _CP - 1) // (16 * _CP)
    share = n_steps * _CP

    def prep_step(s, carry):
        r0 = sid * share + s * _CP
        r0 = jnp.minimum(r0, _NROW_PAD - _CP)
        pltpu.sync_copy(emb.at[pl.ds(r0, _CP)], itlb.at[pl.ds(0, _CP)])
        pltpu.sync_copy(emb.at[pl.ds(_N_TOTAL - 4 + r0, _CP + 8)],
                        itlb.at[pl.ds(_CP, _CP + 8)])

        def ig(g, c2):
            p = g * 16 + iota
            v0 = itlb[pl.ds(g * 16, 16)]
            v1 = plsc.load_gather(itlb, [p + (_CP + 4)])
            plsc.store_scatter(vals0, [p, jnp.full((16,), 0, jnp.int32)], v0)
            plsc.store_scatter(vals0, [p, jnp.full((16,), 1, jnp.int32)], v1)
            return c2

        lax.fori_loop(0, _CP // 16, ig, 0)
        pltpu.sync_copy(vals0.at[pl.ds(0, _CP)],
                        out_tab.at[pl.ds(region + r0, _CP)])
        return carry

    lax.fori_loop(0, n_steps, prep_step, 0)
    plsc.subcore_barrier()

    def chunk_body(ch, carry):
        base = wid * _PW + ch * _C
        pltpu.sync_copy(xcat.at[pl.ds(base, _C)], xb)
        pltpu.sync_copy(xcat.at[pl.ds(_B + base, _C)], yb)
        pltpu.sync_copy(xcat.at[pl.ds(2 * _B + base, _C)], zb)

        def fused(g, c2):
            s16 = pl.ds(g * 16, 16)
            x = xb[s16]
            y = yb[s16]
            z = zb[s16]
            for l in range(_N_RES_LEVELS):
                gx, fx = _prep(x, _RES[l])
                gy, fy = _prep(y, _RES[l])
                gz, fz = _prep(z, _RES[l])
                hs = _hash_corners(gx, gy, gz)
                ws = _corner_w(fx, fy, fz)
                acc0 = jnp.zeros((16,), jnp.float32)
                acc1 = jnp.zeros((16,), jnp.float32)
                for k in range(8):
                    hidx = _mod_level(hs[k], _SIZES[l], _OFF[l])
                    v0 = plsc.load_gather(tab, [hidx])
                    v1 = plsc.load_gather(tab, [hidx + _TAB_P1])
                    acc0 = acc0 + ws[k] * v0
                    acc1 = acc1 + ws[k] * v1
                ob[2 * l, s16] = acc0
                ob[2 * l + 1, s16] = acc1
            return c2

        lax.fori_loop(0, _G, fused, 0)

        def make_p1(l):
            res = _RES[l]
            off = _OFF[l]
            size = _SIZES[l]
            fxb, fyb, fzb = fbufs[l % 2]
            idxr = ibufs[l % 2]

            def p1(g, c2):
                s16 = pl.ds(g * 16, 16)
                s16o = g * 16
                gx, fx = _prep(xb[s16], res)
                gy, fy = _prep(yb[s16], res)
                gz, fz = _prep(zb[s16], res)
                fxb[s16] = fx
                fyb[s16] = fy
                fzb[s16] = fz
                hs = _hash_corners(gx, gy, gz)
                for k in range(8):
                    hidx = _mod_level(hs[k], size, off) + region
                    idxr[pl.ds(k * _C + s16o, 16)] = hidx
                return c2

            return p1

        def make_p2(l):
            fxb, fyb, fzb = fbufs[l % 2]
            vals = vbufs[l % 2]

            def p2(g, c2):
                s16 = pl.ds(g * 16, 16)
                ws = _corner_w(fxb[s16], fyb[s16], fzb[s16])
                acc0 = jnp.zeros((16,), jnp.float32)
                acc1 = jnp.zeros((16,), jnp.float32)
                r0 = g * 16 + iota
                c0 = jnp.full((16,), 0, jnp.int32)
                c1 = jnp.full((16,), 1, jnp.int32)
                for k in range(8):
                    v0 = plsc.load_gather(vals, [r0 + k * _C, c0])
                    v1 = plsc.load_gather(vals, [r0 + k * _C, c1])
                    acc0 = acc0 + ws[k] * v0
                    acc1 = acc1 + ws[k] * v1
                ob[2 * l, s16] = acc0
                ob[2 * l + 1, s16] = acc1
                return c2

            return p2

        def start_gather(l):
            return pltpu.async_copy(out_tab.at[ibufs[l % 2]], vbufs[l % 2],
                                    sems[l % 2])

        l0 = _N_RES_LEVELS
        lax.fori_loop(0, _G, make_p1(l0), 0)
        handle = start_gather(l0)
        for l in range(l0, _N_LEVELS):
            nxt = None
            if l + 1 < _N_LEVELS:
                lax.fori_loop(0, _G, make_p1(l + 1), 0)
                nxt = start_gather(l + 1)
            handle.wait()
            lax.fori_loop(0, _G, make_p2(l), 0)
            handle = nxt

        for c in range(_N_PLANES):
            pltpu.sync_copy(ob.at[c], outs[c].at[pl.ds(base, _C)])
        return carry

    lax.fori_loop(0, _NCH, chunk_body, 0)


_sc_call = pl.kernel(
    _body,
    out_type=([jax.ShapeDtypeStruct((_B,), jnp.float32)] * _N_PLANES
              + [jax.ShapeDtypeStruct((2 * _NROW_PAD, _F), jnp.float32)]),
    mesh=plsc.VectorSubcoreMesh(core_axis_name="c", subcore_axis_name="s"),
    compiler_params=pltpu.CompilerParams(
        needs_layout_passes=False, use_tc_tiling_on_sc=False),
    scratch_types=[
        pltpu.VMEM((_C,), jnp.float32),
        pltpu.VMEM((_C,), jnp.float32),
        pltpu.VMEM((_C,), jnp.float32),
        pltpu.VMEM((_C,), jnp.float32),
        pltpu.VMEM((_C,), jnp.float32),
        pltpu.VMEM((_C,), jnp.float32),
        pltpu.VMEM((_C,), jnp.float32),
        pltpu.VMEM((_C,), jnp.float32),
        pltpu.VMEM((_C,), jnp.float32),
        pltpu.VMEM((8 * _C,), jnp.int32),
        pltpu.VMEM((8 * _C,), jnp.int32),
        pltpu.VMEM((8 * _C, _F), jnp.float32),
        pltpu.VMEM((8 * _C, _F), jnp.float32),
        pltpu.VMEM((_TAB_WORDS,), jnp.float32),
        pltpu.VMEM((2 * _CP + 8,), jnp.float32),
        pltpu.VMEM((_N_PLANES, _C), jnp.float32),
        pltpu.SemaphoreType.DMA,
        pltpu.SemaphoreType.DMA,
    ],
)


def _assemble_body(*refs):
    in_refs = refs[:_OUT_D]
    o_ref = refs[_OUT_D]
    o_ref[...] = jnp.stack([r[...] for r in in_refs])


_assemble = pl.pallas_call(
    _assemble_body,
    grid=(_B // _TR,),
    in_specs=[pl.BlockSpec((_TR,), lambda i: (i,))] * _OUT_D,
    out_specs=pl.BlockSpec((_OUT_D, _TR), lambda i: (0, i)),
    out_shape=jax.ShapeDtypeStruct((_OUT_D, _B), jnp.float32),
)


@jax.jit
def kernel(xyz, embeddings):
    x = xyz[:, 0]
    y = xyz[:, 1]
    z = xyz[:, 2]
    xcat = jnp.concatenate([x, y, z])
    emb_planes = jnp.concatenate(
        [embeddings[:, 0], embeddings[:, 1],
         jnp.zeros((8,), jnp.float32)])
    res = _sc_call(xcat, emb_planes)
    planes = res[:_N_PLANES]
    out_t = _assemble(x, y, z, *planes)
    return out_t.T


# speedup vs baseline: 3.6366x; 1.0588x over previous
import math

import jax
import jax.numpy as jnp
from jax import lax
from jax.experimental import pallas as pl
from jax.experimental.pallas import tpu as pltpu
from jax.experimental.pallas import tpu_sc as plsc

_N_LEVELS = 16
_F = 2
_T = 2 ** 19
_BASE_RES = 16
_MAX_RES = 512
_B = 524288
_SCALE = math.exp(math.log(_MAX_RES / _BASE_RES) / (_N_LEVELS - 1))
_RES = []
_OFF = []
_tot = 0
for _i in range(_N_LEVELS):
    _OFF.append(_tot)
    _r = math.floor(_BASE_RES * _SCALE ** _i)
    _RES.append(_r)
    _tot += min(_T, (_r + 1) ** 3)
_OFF.append(_tot)
_N_TOTAL = _tot
_SIZES = [_OFF[i + 1] - _OFF[i] for i in range(_N_LEVELS)]
_P1 = 2654435761
_P2 = 805459861
_OUT_D = 3 + 2 * _N_LEVELS
_N_PLANES = 2 * _N_LEVELS

_NW = 32
_PW = _B // _NW
_C = 512
_NCH = _PW // _C
_G = _C // 16

_NROW_PAD = 5262480
_CP = 2048
_C2 = 8 * 512
_N_RES_LEVELS = 2
_TAB_ROWS = _OFF[_N_RES_LEVELS]
_TAB_P1 = 14180
_TAB_WORDS = _TAB_P1 + _TAB_ROWS + 11

_TR = 4096


def _hash_corners(gx, gy, gz):
    hx = (gx, gx + jnp.uint32(1))
    hy0 = gy * jnp.uint32(_P1)
    hy = (hy0, hy0 + jnp.uint32(_P1))
    hz0 = gz * jnp.uint32(_P2)
    hz = (hz0, hz0 + jnp.uint32(_P2))
    hyz = (hy[0] ^ hz[0], hy[0] ^ hz[1], hy[1] ^ hz[0], hy[1] ^ hz[1])
    return tuple(hx[(k >> 2) & 1] ^ hyz[k & 3] for k in range(8))


def _mod_level(h, size, off):
    if size == _T:
        r = h & jnp.uint32(_T - 1)
    else:
        hf = h.astype(jnp.float32)
        q = (hf * jnp.float32(1.0 / size)).astype(jnp.int32)
        qu = lax.bitcast_convert_type(q, jnp.uint32)
        r = h - qu * jnp.uint32(size)
        ri = lax.bitcast_convert_type(r, jnp.int32)
        r = jnp.where(ri < 0, r + jnp.uint32(size), r)
        r = jnp.where(r >= jnp.uint32(size), r - jnp.uint32(size), r)
    return lax.bitcast_convert_type(r, jnp.int32) + off


def _prep(v, res):
    vn = jnp.minimum(jnp.maximum(v, jnp.float32(0.0)), jnp.float32(1.0))
    pos = vn * jnp.float32(res)
    gi = pos.astype(jnp.int32)
    gi = jnp.minimum(gi, jnp.int32(res - 1))
    fr = pos - gi.astype(jnp.float32)
    return gi.astype(jnp.uint32), fr


def _corner_w(fx, fy, fz):
    one = jnp.float32(1.0)
    wx = (one - fx, fx)
    wy = (one - fy, fy)
    wz = (one - fz, fz)
    return tuple((wx[(k >> 2) & 1] * wy[(k >> 1) & 1]) * wz[k & 1]
                 for k in range(8))


def _body(*refs):
    xcat, emb = refs[0:2]
    outs = refs[2:2 + _N_PLANES]
    out_tab = refs[2 + _N_PLANES]
    (xb, yb, zb, f0x, f0y, f0z, f1x, f1y, f1z,
     idx0, idx1, vals0, vals1, tab, itlb, ob, sem0, sem1) = refs[3 + _N_PLANES:]
    cid = lax.axis_index("c")
    sid = lax.axis_index("s")
    wid = sid * 2 + cid
    iota = lax.iota(jnp.int32, 16)
    one_i = jnp.full((16,), 1, jnp.int32)
    fbufs = ((f0x, f0y, f0z), (f1x, f1y, f1z))
    ibufs = (idx0, idx1)
    vbufs = (vals0, vals1)
    sems = (sem0, sem1)

    pltpu.sync_copy(emb.at[pl.ds(0, _TAB_ROWS)], tab.at[pl.ds(0, _TAB_ROWS)])
    pltpu.sync_copy(emb.at[pl.ds(_N_TOTAL - 4, _TAB_ROWS + 8)],
                    tab.at[pl.ds(_TAB_P1 - 4, _TAB_ROWS + 8)])

    region = cid * _NROW_PAD
    n_steps = (_N_TOTAL + 16 * _CP - 1) // (16 * _CP)
    share = n_steps * _CP

    def prep_step(s, carry):
        r0 = sid * share + s * _CP
        r0 = jnp.minimum(r0, _NROW_PAD - _CP)
        pltpu.sync_copy(emb.at[pl.ds(r0, _CP)], itlb.at[pl.ds(0, _CP)])
        pltpu.sync_copy(emb.at[pl.ds(_N_TOTAL - 4 + r0, _CP + 8)],
                        itlb.at[pl.ds(_CP, _CP + 8)])

        def ig(g, c2):
            p = g * 16 + iota
            v0 = itlb[pl.ds(g * 16, 16)]
            v1 = plsc.load_gather(itlb, [p + (_CP + 4)])
            plsc.store_scatter(vals0, [p, jnp.full((16,), 0, jnp.int32)], v0)
            plsc.store_scatter(vals0, [p, jnp.full((16,), 1, jnp.int32)], v1)
            return c2

        lax.fori_loop(0, _CP // 16, ig, 0)
        pltpu.sync_copy(vals0.at[pl.ds(0, _CP)],
                        out_tab.at[pl.ds(region + r0, _CP)])
        return carry

    lax.fori_loop(0, n_steps, prep_step, 0)
    plsc.subcore_barrier()

    def chunk_body(ch, carry):
        base = wid * _PW + ch * _C
        pltpu.sync_copy(xcat.at[pl.ds(base, _C)], xb)
        pltpu.sync_copy(xcat.at[pl.ds(_B + base, _C)], yb)
        pltpu.sync_copy(xcat.at[pl.ds(2 * _B + base, _C)], zb)

        def fused(g, c2):
            s16 = pl.ds(g * 16, 16)
            x = xb[s16]
            y = yb[s16]
            z = zb[s16]
            for l in range(_N_RES_LEVELS):
                gx, fx = _prep(x, _RES[l])
                gy, fy = _prep(y, _RES[l])
                gz, fz = _prep(z, _RES[l])
                hs = _hash_corners(gx, gy, gz)
                ws = _corner_w(fx, fy, fz)
                acc0 = jnp.zeros((16,), jnp.float32)
                acc1 = jnp.zeros((16,), jnp.float32)
                for k in range(8):
                    hidx = _mod_level(hs[k], _SIZES[l], _OFF[l])
                    v0 = plsc.load_gather(tab, [hidx])
                    v1 = plsc.load_gather(tab, [hidx + _TAB_P1])
                    acc0 = acc0 + ws[k] * v0
                    acc1 = acc1 + ws[k] * v1
                ob[2 * l, s16] = acc0
                ob[2 * l + 1, s16] = acc1
            return c2

        lax.fori_loop(0, _G, fused, 0)

        def make_p1(l):
            res = _RES[l]
            off = _OFF[l]
            size = _SIZES[l]
            fxb, fyb, fzb = fbufs[l % 2]
            idxr = ibufs[l % 2]

            def p1(g, c2):
                s16 = pl.ds(g * 16, 16)
                s16o = g * 16
                gx, fx = _prep(xb[s16], res)
                gy, fy = _prep(yb[s16], res)
                gz, fz = _prep(zb[s16], res)
                fxb[s16] = fx
                fyb[s16] = fy
                fzb[s16] = fz
                hs = _hash_corners(gx, gy, gz)
                for k in range(8):
                    hidx = _mod_level(hs[k], size, off) + region
                    idxr[pl.ds(k * _C + s16o, 16)] = hidx
                return c2

            return p1

        def make_p2(l):
            fxb, fyb, fzb = fbufs[l % 2]
            vals = vbufs[l % 2]

            def p2(g, c2):
                s16 = pl.ds(g * 16, 16)
                ws = _corner_w(fxb[s16], fyb[s16], fzb[s16])
                acc0 = jnp.zeros((16,), jnp.float32)
                acc1 = jnp.zeros((16,), jnp.float32)
                r0 = g * 16 + iota
                c0 = jnp.full((16,), 0, jnp.int32)
                c1 = jnp.full((16,), 1, jnp.int32)
                for k in range(8):
                    v0 = plsc.load_gather(vals, [r0 + k * _C, c0])
                    v1 = plsc.load_gather(vals, [r0 + k * _C, c1])
                    acc0 = acc0 + ws[k] * v0
                    acc1 = acc1 + ws[k] * v1
                ob[2 * l, s16] = acc0
                ob[2 * l + 1, s16] = acc1
                return c2

            return p2

        def start_gather(l):
            return pltpu.async_copy(out_tab.at[ibufs[l % 2]], vbufs[l % 2],
                                    sems[l % 2])

        l0 = _N_RES_LEVELS
        lax.fori_loop(0, _G, make_p1(l0), 0)
        handle = start_gather(l0)
        for l in range(l0, _N_LEVELS):
            nxt = None
            if l + 1 < _N_LEVELS:
                lax.fori_loop(0, _G, make_p1(l + 1), 0)
                nxt = start_gather(l + 1)
            handle.wait()
            lax.fori_loop(0, _G, make_p2(l), 0)
            handle = nxt

        for c in range(_N_PLANES):
            pltpu.sync_copy(ob.at[c], outs[c].at[pl.ds(base, _C)])
        return carry

    lax.fori_loop(0, _NCH, chunk_body, 0)


_sc_call = pl.kernel(
    _body,
    out_type=([jax.ShapeDtypeStruct((_B,), jnp.float32)] * _N_PLANES
              + [jax.ShapeDtypeStruct((2 * _NROW_PAD, _F), jnp.float32)]),
    mesh=plsc.VectorSubcoreMesh(core_axis_name="c", subcore_axis_name="s"),
    compiler_params=pltpu.CompilerParams(
        needs_layout_passes=False, use_tc_tiling_on_sc=False),
    scratch_types=[
        pltpu.VMEM((_C,), jnp.float32),
        pltpu.VMEM((_C,), jnp.float32),
        pltpu.VMEM((_C,), jnp.float32),
        pltpu.VMEM((_C,), jnp.float32),
        pltpu.VMEM((_C,), jnp.float32),
        pltpu.VMEM((_C,), jnp.float32),
        pltpu.VMEM((_C,), jnp.float32),
        pltpu.VMEM((_C,), jnp.float32),
        pltpu.VMEM((_C,), jnp.float32),
        pltpu.VMEM((8 * _C,), jnp.int32),
        pltpu.VMEM((8 * _C,), jnp.int32),
        pltpu.VMEM((8 * _C, _F), jnp.float32),
        pltpu.VMEM((8 * _C, _F), jnp.float32),
        pltpu.VMEM((_TAB_WORDS,), jnp.float32),
        pltpu.VMEM((2 * _CP + 8,), jnp.float32),
        pltpu.VMEM((_N_PLANES, _C), jnp.float32),
        pltpu.SemaphoreType.DMA,
        pltpu.SemaphoreType.DMA,
    ],
)


def _assemble_body(*refs):
    in_refs = refs[:_OUT_D]
    o_ref = refs[_OUT_D]
    o_ref[...] = jnp.stack([r[...] for r in in_refs])


_assemble = pl.pallas_call(
    _assemble_body,
    grid=(_B // _TR,),
    in_specs=[pl.BlockSpec((_TR,), lambda i: (i,))] * _OUT_D,
    out_specs=pl.BlockSpec((_OUT_D, _TR), lambda i: (0, i)),
    out_shape=jax.ShapeDtypeStruct((_OUT_D, _B), jnp.float32),
)


@jax.jit
def kernel(xyz, embeddings):
    x = xyz[:, 0]
    y = xyz[:, 1]
    z = xyz[:, 2]
    xcat = jnp.concatenate([x, y, z])
    emb_planes = jnp.concatenate(
        [embeddings[:, 0], embeddings[:, 1],
         jnp.zeros((8,), jnp.float32)])
    res = _sc_call(xcat, emb_planes)
    planes = res[:_N_PLANES]
    out_t = _assemble(x, y, z, *planes)
    return out_t.T
